# baseline (speedup 1.0000x reference)
import jax
import jax.numpy as jnp
from jax.experimental import pallas as pl
from jax.experimental.pallas import tpu as pltpu

N = 4096
DO = 64
BM = 256


def _dot(a, b):
    return jax.lax.dot_general(
        a, b, (((1,), (0,)), ((), ())),
        preferred_element_type=jnp.float32)


def _enc_body(params_ref, a_s_ref, a_f_ref, feat_ref, w_enc_ref,
              emb_ref, y_ref, bias_ref):
    i = pl.program_id(0)

    @pl.when(i == 0)
    def _init():
        y = _dot(feat_ref[...], w_enc_ref[...])
        y_ref[...] = y
        bias = params_ref[2] * jnp.sum(y, axis=0, keepdims=True)
        bias_ref[...] = jnp.broadcast_to(bias, (8, DO))

    y = y_ref[...]
    emb_ref[...] = (params_ref[0] * _dot(a_s_ref[...], y)
                    + params_ref[1] * _dot(a_f_ref[...], y)
                    + bias_ref[0:1, :])


def _enc3_body(params_ref, a_s_ref, a_f_ref, feat_ref, w_enc_ref,
               emb1_ref, emb2_ref,
               w_fc1_ref, b_fc1_ref, w_fc2_ref, b_fc2_ref,
               emb_ref, comb_ref, y_ref, bias_ref):
    i = pl.program_id(0)

    @pl.when(i == 0)
    def _init():
        y = _dot(feat_ref[...], w_enc_ref[...])
        y_ref[...] = y
        bias = params_ref[2] * jnp.sum(y, axis=0, keepdims=True)
        bias_ref[...] = jnp.broadcast_to(bias, (8, DO))

    y = y_ref[...]
    emb3 = (params_ref[0] * _dot(a_s_ref[...], y)
            + params_ref[1] * _dot(a_f_ref[...], y)
            + bias_ref[0:1, :])
    emb_ref[...] = emb3
    t = (_dot(emb1_ref[...], w_fc1_ref[0:DO, :])
         + _dot(emb2_ref[...], w_fc1_ref[DO:2 * DO, :])
         + _dot(emb3, w_fc1_ref[2 * DO:3 * DO, :])
         + b_fc1_ref[...])
    comb_ref[...] = _dot(t, w_fc2_ref[...]) + b_fc2_ref[...]


def _dec_body(comb_ref, w_dec1_ref, w_dec2_ref, w_dec3_ref,
              a_s1_ref, a_s2_ref, a_s3_ref,
              rec1_ref, rec2_ref, rec3_ref,
              d1_ref, d2_ref, d3_ref):
    i = pl.program_id(0)

    @pl.when(i == 0)
    def _init():
        comb = comb_ref[...]
        d1_ref[...] = _dot(comb, w_dec1_ref[...])
        d2_ref[...] = _dot(comb, w_dec2_ref[...])
        d3_ref[...] = _dot(comb, w_dec3_ref[...])

    rec1_ref[...] = _dot(a_s1_ref[...], d1_ref[...])
    rec2_ref[...] = _dot(a_s2_ref[...], d2_ref[...])
    rec3_ref[...] = _dot(a_s3_ref[...], d3_ref[...])


def _row_spec(bm, ncols):
    return pl.BlockSpec((bm, ncols), lambda i: (i, 0))


def _full_spec(shape):
    return pl.BlockSpec(shape, lambda i: tuple(0 for _ in shape))


def _encode_one(params, a_s, a_f, feat, w_enc, d_in):
    return pl.pallas_call(
        _enc_body,
        grid=(N // BM,),
        in_specs=[
            pl.BlockSpec(memory_space=pltpu.SMEM),
            _row_spec(BM, N),
            _row_spec(BM, N),
            _full_spec((N, d_in)),
            _full_spec((d_in, DO)),
        ],
        out_specs=_row_spec(BM, DO),
        out_shape=jax.ShapeDtypeStruct((N, DO), jnp.float32),
        scratch_shapes=[pltpu.VMEM((N, DO), jnp.float32),
                        pltpu.VMEM((8, DO), jnp.float32)],
        compiler_params=pltpu.CompilerParams(
            dimension_semantics=("arbitrary",)),
    )(params, a_s, a_f, feat, w_enc)


def _encode_three(params, a_s, a_f, feat, w_enc, d_in,
                  emb1, emb2, w_fc1, b_fc1, w_fc2, b_fc2):
    return pl.pallas_call(
        _enc3_body,
        grid=(N // BM,),
        in_specs=[
            pl.BlockSpec(memory_space=pltpu.SMEM),
            _row_spec(BM, N),
            _row_spec(BM, N),
            _full_spec((N, d_in)),
            _full_spec((d_in, DO)),
            _row_spec(BM, DO),
            _row_spec(BM, DO),
            _full_spec((3 * DO, DO)), _full_spec((1, DO)),
            _full_spec((DO, DO)), _full_spec((1, DO)),
        ],
        out_specs=[_row_spec(BM, DO), _row_spec(BM, DO)],
        out_shape=[jax.ShapeDtypeStruct((N, DO), jnp.float32),
                   jax.ShapeDtypeStruct((N, DO), jnp.float32)],
        scratch_shapes=[pltpu.VMEM((N, DO), jnp.float32),
                        pltpu.VMEM((8, DO), jnp.float32)],
        compiler_params=pltpu.CompilerParams(
            dimension_semantics=("arbitrary",)),
    )(params, a_s, a_f, feat, w_enc, emb1, emb2,
      w_fc1, b_fc1, w_fc2, b_fc2)


def _decode(comb, w_dec1, w_dec2, w_dec3, a_s1, a_s2, a_s3, d1, d2, d3):
    return pl.pallas_call(
        _dec_body,
        grid=(N // BM,),
        in_specs=[
            _full_spec((N, DO)),
            _full_spec((DO, d1)), _full_spec((DO, d2)), _full_spec((DO, d3)),
            _row_spec(BM, N), _row_spec(BM, N), _row_spec(BM, N),
        ],
        out_specs=[
            _row_spec(BM, d1), _row_spec(BM, d2), _row_spec(BM, d3),
        ],
        out_shape=[
            jax.ShapeDtypeStruct((N, d1), jnp.float32),
            jax.ShapeDtypeStruct((N, d2), jnp.float32),
            jax.ShapeDtypeStruct((N, d3), jnp.float32),
        ],
        scratch_shapes=[
            pltpu.VMEM((N, d1), jnp.float32),
            pltpu.VMEM((N, d2), jnp.float32),
            pltpu.VMEM((N, d3), jnp.float32),
        ],
        compiler_params=pltpu.CompilerParams(
            dimension_semantics=("arbitrary",)),
    )(comb, w_dec1, w_dec2, w_dec3, a_s1, a_s2, a_s3)


def kernel(features_omics1, features_omics2, features_omics3,
           adj_spatial_omics1, adj_feature_omics1,
           adj_spatial_omics2, adj_feature_omics2,
           adj_spatial_omics3, adj_feature_omics3,
           W_conv1, b_conv1, W_conv2, b_conv2, W_conv3, b_conv3,
           W_enc1, W_enc2, W_enc3, W_dec1, W_dec2, W_dec3,
           W_fc1, b_fc1, W_fc2, b_fc2):
    p1 = jnp.concatenate([W_conv1, b_conv1])
    p2 = jnp.concatenate([W_conv2, b_conv2])
    p3 = jnp.concatenate([W_conv3, b_conv3])

    emb1 = _encode_one(p1, adj_spatial_omics1, adj_feature_omics1,
                       features_omics1, W_enc1, features_omics1.shape[1])
    emb2 = _encode_one(p2, adj_spatial_omics2, adj_feature_omics2,
                       features_omics2, W_enc2, features_omics2.shape[1])
    emb3, comb = _encode_three(
        p3, adj_spatial_omics3, adj_feature_omics3,
        features_omics3, W_enc3, features_omics3.shape[1],
        emb1, emb2,
        W_fc1, b_fc1.reshape(1, DO), W_fc2, b_fc2.reshape(1, DO))

    rec1, rec2, rec3 = _decode(
        comb, W_dec1, W_dec2, W_dec3,
        adj_spatial_omics1, adj_spatial_omics2, adj_spatial_omics3,
        W_dec1.shape[1], W_dec2.shape[1], W_dec3.shape[1])

    return (emb1, emb2, emb3, comb, rec1, rec2, rec3)


# speedup vs baseline: 1.0448x; 1.0448x over previous
import jax
import jax.numpy as jnp
from jax.experimental import pallas as pl
from jax.experimental.pallas import tpu as pltpu

N = 4096
DO = 64
BM = 256


def _dot(a, b):
    return jax.lax.dot_general(
        a, b, (((1,), (0,)), ((), ())),
        preferred_element_type=jnp.float32)


def _enc_body(p1_ref, p2_ref, p3_ref,
              a_s1_ref, a_f1_ref, a_s2_ref, a_f2_ref, a_s3_ref, a_f3_ref,
              f1_ref, f2_ref, f3_ref, w1_ref, w2_ref, w3_ref,
              emb1_ref, emb2_ref, emb3_ref,
              y1_ref, y2_ref, y3_ref, b_ref):
    i = pl.program_id(0)

    @pl.when(i == 0)
    def _init():
        y1 = _dot(f1_ref[...], w1_ref[...])
        y2 = _dot(f2_ref[...], w2_ref[...])
        y3 = _dot(f3_ref[...], w3_ref[...])
        y1_ref[...] = y1
        y2_ref[...] = y2
        y3_ref[...] = y3
        b1 = p1_ref[2] * jnp.sum(y1, axis=0, keepdims=True)
        b2 = p2_ref[2] * jnp.sum(y2, axis=0, keepdims=True)
        b3 = p3_ref[2] * jnp.sum(y3, axis=0, keepdims=True)
        z = jnp.zeros((5, DO), jnp.float32)
        b_ref[...] = jnp.concatenate([b1, b2, b3, z], axis=0)

    emb1_ref[...] = (p1_ref[0] * _dot(a_s1_ref[...], y1_ref[...])
                     + p1_ref[1] * _dot(a_f1_ref[...], y1_ref[...])
                     + b_ref[0:1, :])
    emb2_ref[...] = (p2_ref[0] * _dot(a_s2_ref[...], y2_ref[...])
                     + p2_ref[1] * _dot(a_f2_ref[...], y2_ref[...])
                     + b_ref[1:2, :])
    emb3_ref[...] = (p3_ref[0] * _dot(a_s3_ref[...], y3_ref[...])
                     + p3_ref[1] * _dot(a_f3_ref[...], y3_ref[...])
                     + b_ref[2:3, :])


def _dec_body(emb1_ref, emb2_ref, emb3_ref,
              w_fc1_ref, b_fc1_ref, w_fc2_ref, b_fc2_ref,
              w_dec1_ref, w_dec2_ref, w_dec3_ref,
              a_s1_ref, a_s2_ref, a_s3_ref,
              comb_ref, rec1_ref, rec2_ref, rec3_ref,
              d1_ref, d2_ref, d3_ref):
    i = pl.program_id(0)

    @pl.when(i == 0)
    def _init():
        t = (_dot(emb1_ref[...], w_fc1_ref[0:DO, :])
             + _dot(emb2_ref[...], w_fc1_ref[DO:2 * DO, :])
             + _dot(emb3_ref[...], w_fc1_ref[2 * DO:3 * DO, :])
             + b_fc1_ref[...])
        comb = _dot(t, w_fc2_ref[...]) + b_fc2_ref[...]
        comb_ref[...] = comb
        d1_ref[...] = _dot(comb, w_dec1_ref[...])
        d2_ref[...] = _dot(comb, w_dec2_ref[...])
        d3_ref[...] = _dot(comb, w_dec3_ref[...])

    rec1_ref[...] = _dot(a_s1_ref[...], d1_ref[...])
    rec2_ref[...] = _dot(a_s2_ref[...], d2_ref[...])
    rec3_ref[...] = _dot(a_s3_ref[...], d3_ref[...])


def _row_spec(bm, ncols):
    return pl.BlockSpec((bm, ncols), lambda i: (i, 0))


def _full_spec(shape):
    return pl.BlockSpec(shape, lambda i: tuple(0 for _ in shape))


def _smem_spec():
    return pl.BlockSpec(memory_space=pltpu.SMEM)


def _encode(p1, p2, p3, a_s1, a_f1, a_s2, a_f2, a_s3, a_f3,
            f1, f2, f3, w1, w2, w3):
    d1, d2, d3 = f1.shape[1], f2.shape[1], f3.shape[1]
    return pl.pallas_call(
        _enc_body,
        grid=(N // BM,),
        in_specs=[
            _smem_spec(), _smem_spec(), _smem_spec(),
            _row_spec(BM, N), _row_spec(BM, N), _row_spec(BM, N),
            _row_spec(BM, N), _row_spec(BM, N), _row_spec(BM, N),
            _full_spec((N, d1)), _full_spec((N, d2)), _full_spec((N, d3)),
            _full_spec((d1, DO)), _full_spec((d2, DO)), _full_spec((d3, DO)),
        ],
        out_specs=[
            _row_spec(BM, DO), _row_spec(BM, DO), _row_spec(BM, DO),
        ],
        out_shape=[
            jax.ShapeDtypeStruct((N, DO), jnp.float32),
            jax.ShapeDtypeStruct((N, DO), jnp.float32),
            jax.ShapeDtypeStruct((N, DO), jnp.float32),
        ],
        scratch_shapes=[
            pltpu.VMEM((N, DO), jnp.float32),
            pltpu.VMEM((N, DO), jnp.float32),
            pltpu.VMEM((N, DO), jnp.float32),
            pltpu.VMEM((8, DO), jnp.float32),
        ],
        compiler_params=pltpu.CompilerParams(
            dimension_semantics=("arbitrary",),
            vmem_limit_bytes=63 * 1024 * 1024),
    )(p1, p2, p3, a_s1, a_f1, a_s2, a_f2, a_s3, a_f3,
      f1, f2, f3, w1, w2, w3)


def _decode(emb1, emb2, emb3, w_fc1, b_fc1, w_fc2, b_fc2,
            w_dec1, w_dec2, w_dec3, a_s1, a_s2, a_s3, d1, d2, d3):
    return pl.pallas_call(
        _dec_body,
        grid=(N // BM,),
        in_specs=[
            _full_spec((N, DO)), _full_spec((N, DO)), _full_spec((N, DO)),
            _full_spec((3 * DO, DO)), _full_spec((1, DO)),
            _full_spec((DO, DO)), _full_spec((1, DO)),
            _full_spec((DO, d1)), _full_spec((DO, d2)), _full_spec((DO, d3)),
            _row_spec(BM, N), _row_spec(BM, N), _row_spec(BM, N),
        ],
        out_specs=[
            _full_spec((N, DO)),
            _row_spec(BM, d1), _row_spec(BM, d2), _row_spec(BM, d3),
        ],
        out_shape=[
            jax.ShapeDtypeStruct((N, DO), jnp.float32),
            jax.ShapeDtypeStruct((N, d1), jnp.float32),
            jax.ShapeDtypeStruct((N, d2), jnp.float32),
            jax.ShapeDtypeStruct((N, d3), jnp.float32),
        ],
        scratch_shapes=[
            pltpu.VMEM((N, d1), jnp.float32),
            pltpu.VMEM((N, d2), jnp.float32),
            pltpu.VMEM((N, d3), jnp.float32),
        ],
        compiler_params=pltpu.CompilerParams(
            dimension_semantics=("arbitrary",)),
    )(emb1, emb2, emb3, w_fc1, b_fc1, w_fc2, b_fc2,
      w_dec1, w_dec2, w_dec3, a_s1, a_s2, a_s3)


def kernel(features_omics1, features_omics2, features_omics3,
           adj_spatial_omics1, adj_feature_omics1,
           adj_spatial_omics2, adj_feature_omics2,
           adj_spatial_omics3, adj_feature_omics3,
           W_conv1, b_conv1, W_conv2, b_conv2, W_conv3, b_conv3,
           W_enc1, W_enc2, W_enc3, W_dec1, W_dec2, W_dec3,
           W_fc1, b_fc1, W_fc2, b_fc2):
    p1 = jnp.concatenate([W_conv1, b_conv1])
    p2 = jnp.concatenate([W_conv2, b_conv2])
    p3 = jnp.concatenate([W_conv3, b_conv3])

    emb1, emb2, emb3 = _encode(
        p1, p2, p3,
        adj_spatial_omics1, adj_feature_omics1,
        adj_spatial_omics2, adj_feature_omics2,
        adj_spatial_omics3, adj_feature_omics3,
        features_omics1, features_omics2, features_omics3,
        W_enc1, W_enc2, W_enc3)

    comb, rec1, rec2, rec3 = _decode(
        emb1, emb2, emb3,
        W_fc1, b_fc1.reshape(1, DO), W_fc2, b_fc2.reshape(1, DO),
        W_dec1, W_dec2, W_dec3,
        adj_spatial_omics1, adj_spatial_omics2, adj_spatial_omics3,
        W_dec1.shape[1], W_dec2.shape[1], W_dec3.shape[1])

    return (emb1, emb2, emb3, comb, rec1, rec2, rec3)
